# baseline (speedup 1.0000x reference)
import functools

import jax
import jax.numpy as jnp
from jax import lax
from jax.experimental import pallas as pl
from jax.experimental.pallas import tpu as pltpu
from jax.experimental.pallas import tpu_sc as plsc

N_NODES = 10000
N_EDGES = 320000
D_IN = 128
D_HID = 128
D_OUT = 64

NC = 2
NS = 16
NW = NC * NS
K = 128
NCHUNK = N_EDGES // K
CPW = 80
NCHUNK_PAD = CPW * NW
N_PAD = 10240
ROWS_PER_TILE = N_PAD // NS
PAD_NODE = N_PAD - 1
NBUF = 2
HALF = 40

_mesh = lambda: plsc.VectorSubcoreMesh(core_axis_name="c", subcore_axis_name="s")


def _worker_id():
    return lax.axis_index("s") * NC + lax.axis_index("c")


@functools.partial(
    pl.kernel,
    out_type=jax.ShapeDtypeStruct((NC, 1, N_PAD), jnp.float32),
    mesh=_mesh(),
    scratch_types=[
        pltpu.VMEM((K,), jnp.float32),
        pltpu.VMEM((ROWS_PER_TILE,), jnp.float32),
        pltpu.VMEM((CPW, K), jnp.int32),
        pltpu.VMEM_SHARED((N_PAD,), jnp.float32),
        pltpu.SemaphoreType.DMA,
    ],
)
def _deg_kernel(dst_hbm, deg_hbm, ones_v, zbuf, didx, acc, sem):
    w = _worker_id()
    s_ax = lax.axis_index("s")
    c_ax = lax.axis_index("c")

    for j in range(K // 16):
        ones_v[pl.ds(j * 16, 16)] = jnp.ones((16,), jnp.float32)
    for j in range(ROWS_PER_TILE // 16):
        zbuf[pl.ds(j * 16, 16)] = jnp.zeros((16,), jnp.float32)

    base = s_ax * ROWS_PER_TILE
    pltpu.sync_copy(zbuf, acc.at[pl.ds(base, ROWS_PER_TILE)])
    pltpu.sync_copy(dst_hbm.at[pl.ds(w * CPW, CPW)], didx)
    plsc.subcore_barrier()

    def fire(j, _):
        pltpu.async_copy(ones_v, acc.at[didx.at[j]], sem, add=True)
        return 0
    lax.fori_loop(0, CPW, fire, 0, unroll=False)

    def drain(j, _):
        pltpu.make_async_copy(ones_v, acc.at[didx.at[0]], sem).wait()
        return 0
    lax.fori_loop(0, CPW, drain, 0, unroll=False)

    plsc.subcore_barrier()
    pltpu.sync_copy(acc.at[pl.ds(base, ROWS_PER_TILE)],
                    deg_hbm.at[c_ax, 0, pl.ds(base, ROWS_PER_TILE)])


@functools.partial(
    pl.kernel,
    out_type=jax.ShapeDtypeStruct((NC, N_PAD, D_HID), jnp.float32),
    mesh=_mesh(),
    scratch_types=[
        pltpu.VMEM((NBUF, K, D_HID), jnp.float32),
        pltpu.VMEM((HALF, K), jnp.int32),
        pltpu.VMEM((HALF, K), jnp.int32),
        pltpu.VMEM_SHARED((N_PAD, D_HID), jnp.float32),
        [pltpu.SemaphoreType.DMA] * NBUF,
    ],
)
def _agg_kernel(xws_hbm, src_hbm, dst_hbm, agg_hbm, rows, sidx, didx, acc, gsem):
    w = _worker_id()
    s_ax = lax.axis_index("s")
    c_ax = lax.axis_index("c")

    def fz(i, _):
        for j in range(D_HID // 16):
            rows[0, i, pl.ds(j * 16, 16)] = jnp.zeros((16,), jnp.float32)
        return 0
    lax.fori_loop(0, K, fz, 0, unroll=False)
    base = s_ax * ROWS_PER_TILE
    for i in range(ROWS_PER_TILE // K):
        pltpu.sync_copy(rows.at[0], acc.at[pl.ds(base + i * K, K)])
    plsc.subcore_barrier()

    for h in range(CPW // HALF):
        hb = w * CPW + h * HALF
        pltpu.sync_copy(src_hbm.at[pl.ds(hb, HALF)], sidx)
        pltpu.sync_copy(dst_hbm.at[pl.ds(hb, HALF)], didx)
        def body(j, _):
            pltpu.async_copy(xws_hbm.at[sidx.at[j]], rows.at[0], gsem[0]).wait()
            pltpu.sync_copy(rows.at[0], acc.at[didx.at[j]], add=True)
            return 0
        lax.fori_loop(0, HALF, body, 0, unroll=False)

    plsc.subcore_barrier()
    for i in range(ROWS_PER_TILE // K):
        pltpu.sync_copy(acc.at[pl.ds(base + i * K, K)],
                        agg_hbm.at[c_ax, pl.ds(base + i * K, K)])


RBX = 1024


def _xws_body(x_ref, w_ref, deg_ref, o_ref):
    deg = deg_ref[0, :, 0:1] + deg_ref[1, :, 0:1] + 1.0
    dinv = lax.rsqrt(deg)
    xw = jnp.dot(x_ref[...], w_ref[...], preferred_element_type=jnp.float32)
    o_ref[...] = xw * dinv


def _xws_call(x_pad, W_g, deg_parts):
    return pl.pallas_call(
        _xws_body,
        grid=(N_PAD // RBX,),
        in_specs=[
            pl.BlockSpec((RBX, D_IN), lambda i: (i, 0)),
            pl.BlockSpec((D_IN, D_HID), lambda i: (0, 0)),
            pl.BlockSpec((NC, RBX, 1), lambda i: (0, i, 0)),
        ],
        out_specs=pl.BlockSpec((RBX, D_HID), lambda i: (i, 0)),
        out_shape=jax.ShapeDtypeStruct((N_PAD, D_HID), jnp.float32),
    )(x_pad, W_g, deg_parts)


RBH = 1000


def _head_body(agg_ref, xws_ref, deg_ref, bg_ref, wl_ref, bl_ref, o_ref):
    deg = deg_ref[0, :, 0:1] + deg_ref[1, :, 0:1] + 1.0
    dinv = lax.rsqrt(deg)
    pre = (agg_ref[0] + agg_ref[1] + xws_ref[...]) * dinv + bg_ref[...]
    h = jnp.maximum(pre, 0.0)
    o_ref[...] = jnp.dot(h, wl_ref[...], preferred_element_type=jnp.float32) + bl_ref[...]


def _head_call(agg_parts, xws, deg_parts, b_g, W_l, b_l):
    return pl.pallas_call(
        _head_body,
        grid=(N_NODES // RBH,),
        in_specs=[
            pl.BlockSpec((NC, RBH, D_HID), lambda i: (0, i, 0)),
            pl.BlockSpec((RBH, D_HID), lambda i: (i, 0)),
            pl.BlockSpec((NC, RBH, 1), lambda i: (0, i, 0)),
            pl.BlockSpec((1, D_HID), lambda i: (0, 0)),
            pl.BlockSpec((D_HID, D_OUT), lambda i: (0, 0)),
            pl.BlockSpec((1, D_OUT), lambda i: (0, 0)),
        ],
        out_specs=pl.BlockSpec((RBH, D_OUT), lambda i: (i, 0)),
        out_shape=jax.ShapeDtypeStruct((N_NODES, D_OUT), jnp.float32),
    )(agg_parts, xws, deg_parts, b_g, W_l, b_l)


def kernel(x, edge_index, W_g, b_g, W_l, b_l):
    ei = edge_index.astype(jnp.int32).reshape(2, NCHUNK, K)
    pad = jnp.full((2, NCHUNK_PAD - NCHUNK, K), PAD_NODE, dtype=jnp.int32)
    ei = jnp.concatenate([ei, pad], axis=1)
    src, dst = ei[0], ei[1]
    x_pad = jnp.pad(x, ((0, N_PAD - N_NODES), (0, 0)))
    deg_parts = _deg_kernel(dst).reshape(NC, N_PAD, 1)
    xws = _xws_call(x_pad, W_g, deg_parts)
    agg_parts = _agg_kernel(xws, src, dst)
    return _head_call(agg_parts, xws, deg_parts,
                      b_g.reshape(1, D_HID), W_l, b_l.reshape(1, D_OUT))


# speedup vs baseline: 2.4194x; 2.4194x over previous
import functools

import jax
import jax.numpy as jnp
from jax import lax
from jax.experimental import pallas as pl
from jax.experimental.pallas import tpu as pltpu
from jax.experimental.pallas import tpu_sc as plsc

N_NODES = 10000
N_EDGES = 320000
D_IN = 128
D_HID = 128
D_OUT = 64

NC = 2
NS = 16
NW = NC * NS
K = 128
NCHUNK = N_EDGES // K
CPW = 80
NCHUNK_PAD = CPW * NW
N_PAD = 10240
ROWS_PER_TILE = N_PAD // NS
PAD_NODE = N_PAD - 1
NBUF = 2
HALF = 40

_mesh = lambda: plsc.VectorSubcoreMesh(core_axis_name="c", subcore_axis_name="s")


def _worker_id():
    return lax.axis_index("s") * NC + lax.axis_index("c")


@functools.partial(
    pl.kernel,
    out_type=jax.ShapeDtypeStruct((NC, 1, N_PAD), jnp.float32),
    mesh=_mesh(),
    scratch_types=[
        pltpu.VMEM((K,), jnp.float32),
        pltpu.VMEM((ROWS_PER_TILE,), jnp.float32),
        pltpu.VMEM((CPW, K), jnp.int32),
        pltpu.VMEM_SHARED((N_PAD,), jnp.float32),
        pltpu.SemaphoreType.DMA,
    ],
)
def _deg_kernel(dst_hbm, deg_hbm, ones_v, zbuf, didx, acc, sem):
    w = _worker_id()
    s_ax = lax.axis_index("s")
    c_ax = lax.axis_index("c")

    for j in range(K // 16):
        ones_v[pl.ds(j * 16, 16)] = jnp.ones((16,), jnp.float32)
    for j in range(ROWS_PER_TILE // 16):
        zbuf[pl.ds(j * 16, 16)] = jnp.zeros((16,), jnp.float32)

    base = s_ax * ROWS_PER_TILE
    pltpu.sync_copy(zbuf, acc.at[pl.ds(base, ROWS_PER_TILE)])
    pltpu.sync_copy(dst_hbm.at[pl.ds(w * CPW, CPW)], didx)
    plsc.subcore_barrier()

    def fire(j, _):
        pltpu.async_copy(ones_v, acc.at[didx.at[j]], sem, add=True)
        return 0
    lax.fori_loop(0, CPW, fire, 0, unroll=False)

    def drain(j, _):
        pltpu.make_async_copy(ones_v, acc.at[didx.at[0]], sem).wait()
        return 0
    lax.fori_loop(0, CPW, drain, 0, unroll=False)

    plsc.subcore_barrier()
    pltpu.sync_copy(acc.at[pl.ds(base, ROWS_PER_TILE)],
                    deg_hbm.at[c_ax, 0, pl.ds(base, ROWS_PER_TILE)])


@functools.partial(
    pl.kernel,
    out_type=jax.ShapeDtypeStruct((NC, N_PAD, D_HID), jnp.float32),
    mesh=_mesh(),
    scratch_types=[
        pltpu.VMEM((NBUF, K, D_HID), jnp.float32),
        pltpu.VMEM((HALF, K), jnp.int32),
        pltpu.VMEM((HALF, K), jnp.int32),
        pltpu.VMEM_SHARED((N_PAD, D_HID), jnp.float32),
        [pltpu.SemaphoreType.DMA] * NBUF,
    ],
)
def _agg_kernel(xws_hbm, src_hbm, dst_hbm, agg_hbm, rows, sidx, didx, acc, gsem):
    w = _worker_id()
    s_ax = lax.axis_index("s")
    c_ax = lax.axis_index("c")

    def fz(i, _):
        for j in range(D_HID // 16):
            rows[0, i, pl.ds(j * 16, 16)] = jnp.zeros((16,), jnp.float32)
        return 0
    lax.fori_loop(0, K, fz, 0, unroll=False)
    base = s_ax * ROWS_PER_TILE
    for i in range(ROWS_PER_TILE // K):
        pltpu.sync_copy(rows.at[0], acc.at[pl.ds(base + i * K, K)])
    plsc.subcore_barrier()

    for h in range(CPW // HALF):
        hb = w * CPW + h * HALF
        pltpu.sync_copy(src_hbm.at[pl.ds(hb, HALF)], sidx)
        pltpu.sync_copy(dst_hbm.at[pl.ds(hb, HALF)], didx)
        def body(j, _):
            pltpu.async_copy(xws_hbm.at[sidx.at[j]], rows.at[0], gsem[0]).wait()
            pltpu.sync_copy(rows.at[0], acc.at[didx.at[j]], add=True)
            return 0
        lax.fori_loop(0, HALF, body, 0, unroll=False)

    plsc.subcore_barrier()
    for i in range(ROWS_PER_TILE // K):
        pltpu.sync_copy(acc.at[pl.ds(base + i * K, K)],
                        agg_hbm.at[c_ax, pl.ds(base + i * K, K)])


RBX = 1024


def _xws_body(x_ref, w_ref, deg_ref, o_ref):
    deg = deg_ref[0, :, 0:1] + deg_ref[1, :, 0:1] + 1.0
    dinv = lax.rsqrt(deg)
    xw = jnp.dot(x_ref[...], w_ref[...], preferred_element_type=jnp.float32)
    o_ref[...] = xw * dinv


def _xws_call(x_pad, W_g, deg_parts):
    return pl.pallas_call(
        _xws_body,
        grid=(N_PAD // RBX,),
        in_specs=[
            pl.BlockSpec((RBX, D_IN), lambda i: (i, 0)),
            pl.BlockSpec((D_IN, D_HID), lambda i: (0, 0)),
            pl.BlockSpec((NC, RBX, 1), lambda i: (0, i, 0)),
        ],
        out_specs=pl.BlockSpec((RBX, D_HID), lambda i: (i, 0)),
        out_shape=jax.ShapeDtypeStruct((N_PAD, D_HID), jnp.float32),
    )(x_pad, W_g, deg_parts)


RBH = 1000


def _head_body(agg_ref, xws_ref, deg_ref, bg_ref, wl_ref, bl_ref, o_ref):
    deg = deg_ref[0, :, 0:1] + deg_ref[1, :, 0:1] + 1.0
    dinv = lax.rsqrt(deg)
    pre = (agg_ref[0] + agg_ref[1] + xws_ref[...]) * dinv + bg_ref[...]
    h = jnp.maximum(pre, 0.0)
    o_ref[...] = jnp.dot(h, wl_ref[...], preferred_element_type=jnp.float32) + bl_ref[...]


def _head_call(agg_parts, xws, deg_parts, b_g, W_l, b_l):
    return pl.pallas_call(
        _head_body,
        grid=(N_NODES // RBH,),
        in_specs=[
            pl.BlockSpec((NC, RBH, D_HID), lambda i: (0, i, 0)),
            pl.BlockSpec((RBH, D_HID), lambda i: (i, 0)),
            pl.BlockSpec((NC, RBH, 1), lambda i: (0, i, 0)),
            pl.BlockSpec((1, D_HID), lambda i: (0, 0)),
            pl.BlockSpec((D_HID, D_OUT), lambda i: (0, 0)),
            pl.BlockSpec((1, D_OUT), lambda i: (0, 0)),
        ],
        out_specs=pl.BlockSpec((RBH, D_OUT), lambda i: (i, 0)),
        out_shape=jax.ShapeDtypeStruct((N_NODES, D_OUT), jnp.float32),
    )(agg_parts, xws, deg_parts, b_g, W_l, b_l)


def kernel(x, edge_index, W_g, b_g, W_l, b_l):
    ei = edge_index.astype(jnp.int32).reshape(2, NCHUNK, K)
    npadc = NCHUNK_PAD - NCHUNK
    padv = (N_NODES +
            jnp.arange(npadc * K, dtype=jnp.int32) % (N_PAD - N_NODES))
    pad = jnp.broadcast_to(padv.reshape(1, npadc, K), (2, npadc, K))
    ei = jnp.concatenate([ei, pad], axis=1)
    src, dst = ei[0], ei[1]
    x_pad = jnp.pad(x, ((0, N_PAD - N_NODES), (0, 0)))
    deg_parts = _deg_kernel(dst).reshape(NC, N_PAD, 1)
    xws = _xws_call(x_pad, W_g, deg_parts)
    agg_parts = _agg_kernel(xws, src, dst)
    return _head_call(agg_parts, xws, deg_parts,
                      b_g.reshape(1, D_HID), W_l, b_l.reshape(1, D_OUT))


# speedup vs baseline: 3.3426x; 1.3816x over previous
import functools

import jax
import jax.numpy as jnp
from jax import lax
from jax.experimental import pallas as pl
from jax.experimental.pallas import tpu as pltpu
from jax.experimental.pallas import tpu_sc as plsc

N_NODES = 10000
N_EDGES = 320000
D_IN = 128
D_HID = 128
D_OUT = 64

NC = 2
NS = 16
NW = NC * NS
K = 128
NCHUNK = N_EDGES // K
CPW = 80
NCHUNK_PAD = CPW * NW
N_PAD = 10240
ROWS_PER_TILE = N_PAD // NS
PAD_NODE = N_PAD - 1
NBUF = 2
HALF = 40

_mesh = lambda: plsc.VectorSubcoreMesh(core_axis_name="c", subcore_axis_name="s")


def _worker_id():
    return lax.axis_index("s") * NC + lax.axis_index("c")


@functools.partial(
    pl.kernel,
    out_type=jax.ShapeDtypeStruct((NC, 1, N_PAD), jnp.float32),
    mesh=_mesh(),
    scratch_types=[
        pltpu.VMEM((K,), jnp.float32),
        pltpu.VMEM((ROWS_PER_TILE,), jnp.float32),
        pltpu.VMEM((CPW, K), jnp.int32),
        pltpu.VMEM_SHARED((N_PAD,), jnp.float32),
        pltpu.SemaphoreType.DMA,
    ],
)
def _deg_kernel(dst_hbm, deg_hbm, ones_v, zbuf, didx, acc, sem):
    w = _worker_id()
    s_ax = lax.axis_index("s")
    c_ax = lax.axis_index("c")

    for j in range(K // 16):
        ones_v[pl.ds(j * 16, 16)] = jnp.ones((16,), jnp.float32)
    for j in range(ROWS_PER_TILE // 16):
        zbuf[pl.ds(j * 16, 16)] = jnp.zeros((16,), jnp.float32)

    base = s_ax * ROWS_PER_TILE
    pltpu.sync_copy(zbuf, acc.at[pl.ds(base, ROWS_PER_TILE)])
    pltpu.sync_copy(dst_hbm.at[pl.ds(w * CPW, CPW)], didx)
    plsc.subcore_barrier()

    def fire(j, _):
        pltpu.async_copy(ones_v, acc.at[didx.at[j]], sem, add=True)
        return 0
    lax.fori_loop(0, CPW, fire, 0, unroll=False)

    def drain(j, _):
        pltpu.make_async_copy(ones_v, acc.at[didx.at[0]], sem).wait()
        return 0
    lax.fori_loop(0, CPW, drain, 0, unroll=False)

    plsc.subcore_barrier()
    pltpu.sync_copy(acc.at[pl.ds(base, ROWS_PER_TILE)],
                    deg_hbm.at[c_ax, 0, pl.ds(base, ROWS_PER_TILE)])


@functools.partial(
    pl.kernel,
    out_type=jax.ShapeDtypeStruct((NC, N_PAD, D_HID), jnp.float32),
    mesh=_mesh(),
    scratch_types=[
        pltpu.VMEM((NBUF, K, D_HID), jnp.float32),
        pltpu.VMEM((HALF, K), jnp.int32),
        pltpu.VMEM((HALF, K), jnp.int32),
        pltpu.VMEM_SHARED((N_PAD, D_HID), jnp.float32),
        [pltpu.SemaphoreType.DMA] * NBUF,
    ],
)
def _agg_kernel(xws_hbm, src_hbm, dst_hbm, agg_hbm, rows, sidx, didx, acc, gsem):
    w = _worker_id()
    s_ax = lax.axis_index("s")
    c_ax = lax.axis_index("c")

    def fz(i, _):
        for j in range(D_HID // 16):
            rows[0, i, pl.ds(j * 16, 16)] = jnp.zeros((16,), jnp.float32)
        return 0
    lax.fori_loop(0, K, fz, 0, unroll=False)
    base = s_ax * ROWS_PER_TILE
    for i in range(ROWS_PER_TILE // K):
        pltpu.sync_copy(rows.at[0], acc.at[pl.ds(base + i * K, K)])
    plsc.subcore_barrier()

    for h in range(CPW // HALF):
        hb = w * CPW + h * HALF
        pltpu.sync_copy(src_hbm.at[pl.ds(hb, HALF)], sidx)
        pltpu.sync_copy(dst_hbm.at[pl.ds(hb, HALF)], didx)
        for b in range(NBUF):
            pltpu.async_copy(xws_hbm.at[sidx.at[b]], rows.at[b], gsem[b])

        def outer(g, _):
            for b in range(NBUF):
                j = g * NBUF + b
                pltpu.make_async_copy(xws_hbm.at[sidx.at[b]], rows.at[b],
                                      gsem[b]).wait()
                pltpu.sync_copy(rows.at[b], acc.at[didx.at[j]], add=True)
                nxt = j + NBUF
                @pl.when(nxt < HALF)
                def _():
                    pltpu.async_copy(xws_hbm.at[sidx.at[nxt]], rows.at[b],
                                     gsem[b])
            return 0
        lax.fori_loop(0, HALF // NBUF, outer, 0, unroll=False)

    plsc.subcore_barrier()
    for i in range(ROWS_PER_TILE // K):
        pltpu.sync_copy(acc.at[pl.ds(base + i * K, K)],
                        agg_hbm.at[c_ax, pl.ds(base + i * K, K)])


RBX = 1024


def _xws_body(x_ref, w_ref, deg_ref, o_ref):
    deg = deg_ref[0, :, 0:1] + deg_ref[1, :, 0:1] + 1.0
    dinv = lax.rsqrt(deg)
    xw = jnp.dot(x_ref[...], w_ref[...], preferred_element_type=jnp.float32)
    o_ref[...] = xw * dinv


def _xws_call(x_pad, W_g, deg_parts):
    return pl.pallas_call(
        _xws_body,
        grid=(N_PAD // RBX,),
        in_specs=[
            pl.BlockSpec((RBX, D_IN), lambda i: (i, 0)),
            pl.BlockSpec((D_IN, D_HID), lambda i: (0, 0)),
            pl.BlockSpec((NC, RBX, 1), lambda i: (0, i, 0)),
        ],
        out_specs=pl.BlockSpec((RBX, D_HID), lambda i: (i, 0)),
        out_shape=jax.ShapeDtypeStruct((N_PAD, D_HID), jnp.float32),
    )(x_pad, W_g, deg_parts)


RBH = 1000


def _head_body(agg_ref, xws_ref, deg_ref, bg_ref, wl_ref, bl_ref, o_ref):
    deg = deg_ref[0, :, 0:1] + deg_ref[1, :, 0:1] + 1.0
    dinv = lax.rsqrt(deg)
    pre = (agg_ref[0] + agg_ref[1] + xws_ref[...]) * dinv + bg_ref[...]
    h = jnp.maximum(pre, 0.0)
    o_ref[...] = jnp.dot(h, wl_ref[...], preferred_element_type=jnp.float32) + bl_ref[...]


def _head_call(agg_parts, xws, deg_parts, b_g, W_l, b_l):
    return pl.pallas_call(
        _head_body,
        grid=(N_NODES // RBH,),
        in_specs=[
            pl.BlockSpec((NC, RBH, D_HID), lambda i: (0, i, 0)),
            pl.BlockSpec((RBH, D_HID), lambda i: (i, 0)),
            pl.BlockSpec((NC, RBH, 1), lambda i: (0, i, 0)),
            pl.BlockSpec((1, D_HID), lambda i: (0, 0)),
            pl.BlockSpec((D_HID, D_OUT), lambda i: (0, 0)),
            pl.BlockSpec((1, D_OUT), lambda i: (0, 0)),
        ],
        out_specs=pl.BlockSpec((RBH, D_OUT), lambda i: (i, 0)),
        out_shape=jax.ShapeDtypeStruct((N_NODES, D_OUT), jnp.float32),
    )(agg_parts, xws, deg_parts, b_g, W_l, b_l)


def kernel(x, edge_index, W_g, b_g, W_l, b_l):
    ei = edge_index.astype(jnp.int32).reshape(2, NCHUNK, K)
    npadc = NCHUNK_PAD - NCHUNK
    padv = (N_NODES +
            jnp.arange(npadc * K, dtype=jnp.int32) % (N_PAD - N_NODES))
    pad = jnp.broadcast_to(padv.reshape(1, npadc, K), (2, npadc, K))
    ei = jnp.concatenate([ei, pad], axis=1)
    src, dst = ei[0], ei[1]
    x_pad = jnp.pad(x, ((0, N_PAD - N_NODES), (0, 0)))
    deg_parts = _deg_kernel(dst).reshape(NC, N_PAD, 1)
    xws = _xws_call(x_pad, W_g, deg_parts)
    agg_parts = _agg_kernel(xws, src, dst)
    return _head_call(agg_parts, xws, deg_parts,
                      b_g.reshape(1, D_HID), W_l, b_l.reshape(1, D_OUT))


# speedup vs baseline: 3.3910x; 1.0145x over previous
import functools

import jax
import jax.numpy as jnp
from jax import lax
from jax.experimental import pallas as pl
from jax.experimental.pallas import tpu as pltpu
from jax.experimental.pallas import tpu_sc as plsc

N_NODES = 10000
N_EDGES = 320000
D_IN = 128
D_HID = 128
D_OUT = 64

NC = 2
NS = 16
NW = NC * NS
K = 64
NCHUNK = N_EDGES // K
CPW = 160
NCHUNK_PAD = CPW * NW
N_PAD = 10240
ROWS_PER_TILE = N_PAD // NS
PAD_NODE = N_PAD - 1
NBUF = 4
HALF = 40

_mesh = lambda: plsc.VectorSubcoreMesh(core_axis_name="c", subcore_axis_name="s")


def _worker_id():
    return lax.axis_index("s") * NC + lax.axis_index("c")


@functools.partial(
    pl.kernel,
    out_type=jax.ShapeDtypeStruct((NC, 1, N_PAD), jnp.float32),
    mesh=_mesh(),
    scratch_types=[
        pltpu.VMEM((K,), jnp.float32),
        pltpu.VMEM((ROWS_PER_TILE,), jnp.float32),
        pltpu.VMEM((CPW, K), jnp.int32),
        pltpu.VMEM_SHARED((N_PAD,), jnp.float32),
        pltpu.SemaphoreType.DMA,
    ],
)
def _deg_kernel(dst_hbm, deg_hbm, ones_v, zbuf, didx, acc, sem):
    w = _worker_id()
    s_ax = lax.axis_index("s")
    c_ax = lax.axis_index("c")

    for j in range(K // 16):
        ones_v[pl.ds(j * 16, 16)] = jnp.ones((16,), jnp.float32)
    for j in range(ROWS_PER_TILE // 16):
        zbuf[pl.ds(j * 16, 16)] = jnp.zeros((16,), jnp.float32)

    base = s_ax * ROWS_PER_TILE
    pltpu.sync_copy(zbuf, acc.at[pl.ds(base, ROWS_PER_TILE)])
    pltpu.sync_copy(dst_hbm.at[pl.ds(w * CPW, CPW)], didx)
    plsc.subcore_barrier()

    def fire(j, _):
        pltpu.async_copy(ones_v, acc.at[didx.at[j]], sem, add=True)
        return 0
    lax.fori_loop(0, CPW, fire, 0, unroll=False)

    def drain(j, _):
        pltpu.make_async_copy(ones_v, acc.at[didx.at[0]], sem).wait()
        return 0
    lax.fori_loop(0, CPW, drain, 0, unroll=False)

    plsc.subcore_barrier()
    pltpu.sync_copy(acc.at[pl.ds(base, ROWS_PER_TILE)],
                    deg_hbm.at[c_ax, 0, pl.ds(base, ROWS_PER_TILE)])


@functools.partial(
    pl.kernel,
    out_type=jax.ShapeDtypeStruct((NC, N_PAD, D_HID), jnp.float32),
    mesh=_mesh(),
    scratch_types=[
        pltpu.VMEM((NBUF, K, D_HID), jnp.float32),
        pltpu.VMEM((HALF, K), jnp.int32),
        pltpu.VMEM((HALF, K), jnp.int32),
        pltpu.VMEM_SHARED((N_PAD, D_HID), jnp.float32),
        [pltpu.SemaphoreType.DMA] * NBUF,
    ],
)
def _agg_kernel(xws_hbm, src_hbm, dst_hbm, agg_hbm, rows, sidx, didx, acc, gsem):
    w = _worker_id()
    s_ax = lax.axis_index("s")
    c_ax = lax.axis_index("c")

    def fz(i, _):
        for j in range(D_HID // 16):
            rows[0, i, pl.ds(j * 16, 16)] = jnp.zeros((16,), jnp.float32)
        return 0
    lax.fori_loop(0, K, fz, 0, unroll=False)
    base = s_ax * ROWS_PER_TILE
    for i in range(ROWS_PER_TILE // K):
        pltpu.sync_copy(rows.at[0], acc.at[pl.ds(base + i * K, K)])
    plsc.subcore_barrier()

    for h in range(CPW // HALF):
        hb = w * CPW + h * HALF
        pltpu.sync_copy(src_hbm.at[pl.ds(hb, HALF)], sidx)
        pltpu.sync_copy(dst_hbm.at[pl.ds(hb, HALF)], didx)
        for b in range(NBUF):
            pltpu.async_copy(xws_hbm.at[sidx.at[b]], rows.at[b], gsem[b])

        def outer(g, _):
            for b in range(NBUF):
                j = g * NBUF + b
                pltpu.make_async_copy(xws_hbm.at[sidx.at[b]], rows.at[b],
                                      gsem[b]).wait()
                pltpu.sync_copy(rows.at[b], acc.at[didx.at[j]], add=True)
                nxt = j + NBUF
                @pl.when(nxt < HALF)
                def _():
                    pltpu.async_copy(xws_hbm.at[sidx.at[nxt]], rows.at[b],
                                     gsem[b])
            return 0
        lax.fori_loop(0, HALF // NBUF, outer, 0, unroll=False)

    plsc.subcore_barrier()
    for i in range(ROWS_PER_TILE // K):
        pltpu.sync_copy(acc.at[pl.ds(base + i * K, K)],
                        agg_hbm.at[c_ax, pl.ds(base + i * K, K)])


RBX = 1024


def _xws_body(x_ref, w_ref, deg_ref, o_ref):
    deg = deg_ref[0, :, 0:1] + deg_ref[1, :, 0:1] + 1.0
    dinv = lax.rsqrt(deg)
    xw = jnp.dot(x_ref[...], w_ref[...], preferred_element_type=jnp.float32)
    o_ref[...] = xw * dinv


def _xws_call(x_pad, W_g, deg_parts):
    return pl.pallas_call(
        _xws_body,
        grid=(N_PAD // RBX,),
        in_specs=[
            pl.BlockSpec((RBX, D_IN), lambda i: (i, 0)),
            pl.BlockSpec((D_IN, D_HID), lambda i: (0, 0)),
            pl.BlockSpec((NC, RBX, 1), lambda i: (0, i, 0)),
        ],
        out_specs=pl.BlockSpec((RBX, D_HID), lambda i: (i, 0)),
        out_shape=jax.ShapeDtypeStruct((N_PAD, D_HID), jnp.float32),
    )(x_pad, W_g, deg_parts)


RBH = 1000


def _head_body(agg_ref, xws_ref, deg_ref, bg_ref, wl_ref, bl_ref, o_ref):
    deg = deg_ref[0, :, 0:1] + deg_ref[1, :, 0:1] + 1.0
    dinv = lax.rsqrt(deg)
    pre = (agg_ref[0] + agg_ref[1] + xws_ref[...]) * dinv + bg_ref[...]
    h = jnp.maximum(pre, 0.0)
    o_ref[...] = jnp.dot(h, wl_ref[...], preferred_element_type=jnp.float32) + bl_ref[...]


def _head_call(agg_parts, xws, deg_parts, b_g, W_l, b_l):
    return pl.pallas_call(
        _head_body,
        grid=(N_NODES // RBH,),
        in_specs=[
            pl.BlockSpec((NC, RBH, D_HID), lambda i: (0, i, 0)),
            pl.BlockSpec((RBH, D_HID), lambda i: (i, 0)),
            pl.BlockSpec((NC, RBH, 1), lambda i: (0, i, 0)),
            pl.BlockSpec((1, D_HID), lambda i: (0, 0)),
            pl.BlockSpec((D_HID, D_OUT), lambda i: (0, 0)),
            pl.BlockSpec((1, D_OUT), lambda i: (0, 0)),
        ],
        out_specs=pl.BlockSpec((RBH, D_OUT), lambda i: (i, 0)),
        out_shape=jax.ShapeDtypeStruct((N_NODES, D_OUT), jnp.float32),
    )(agg_parts, xws, deg_parts, b_g, W_l, b_l)


def kernel(x, edge_index, W_g, b_g, W_l, b_l):
    ei = edge_index.astype(jnp.int32).reshape(2, NCHUNK, K)
    npadc = NCHUNK_PAD - NCHUNK
    padv = (N_NODES +
            jnp.arange(npadc * K, dtype=jnp.int32) % (N_PAD - N_NODES))
    pad = jnp.broadcast_to(padv.reshape(1, npadc, K), (2, npadc, K))
    ei = jnp.concatenate([ei, pad], axis=1)
    src, dst = ei[0], ei[1]
    x_pad = jnp.pad(x, ((0, N_PAD - N_NODES), (0, 0)))
    deg_parts = _deg_kernel(dst).reshape(NC, N_PAD, 1)
    xws = _xws_call(x_pad, W_g, deg_parts)
    agg_parts = _agg_kernel(xws, src, dst)
    return _head_call(agg_parts, xws, deg_parts,
                      b_g.reshape(1, D_HID), W_l, b_l.reshape(1, D_OUT))
